# baseline (speedup 1.0000x reference)
import functools

import jax
import jax.numpy as jnp
from jax.experimental import pallas as pl
from jax.experimental.pallas import tpu as pltpu

_B, _D, _O, _H, _E, _K = 8192, 1024, 1024, 2048, 8, 2
_TM = 128
_PMAX = _B * _K + _E * _TM
_NTILES = _PMAX // _TM
_TG = 1024


def _gate_body(x_ref, wg_ref, i1_ref, i2_ref, w1_ref, w2_ref):
    s = jnp.dot(x_ref[...], wg_ref[...], preferred_element_type=jnp.float32)
    cols = jax.lax.broadcasted_iota(jnp.int32, s.shape, 1)
    neg = jnp.float32(-jnp.inf)
    s = jnp.where(cols < _E, s, neg)
    m1 = jnp.max(s, axis=1)
    i1 = jnp.min(jnp.where(s == m1[:, None], cols, _E), axis=1)
    s2 = jnp.where(cols == i1[:, None], neg, s)
    m2 = jnp.max(s2, axis=1)
    i2 = jnp.min(jnp.where(s2 == m2[:, None], cols, _E), axis=1)
    d = jnp.exp(m2 - m1)
    i1_ref[...] = i1
    i2_ref[...] = i2
    w1_ref[...] = 1.0 / (1.0 + d)
    w2_ref[...] = d / (1.0 + d)


def _gate(x, wg_pad):
    return pl.pallas_call(
        _gate_body,
        grid=(_B // _TG,),
        in_specs=[
            pl.BlockSpec((_TG, _D), lambda m: (m, 0)),
            pl.BlockSpec((_D, 128), lambda m: (0, 0)),
        ],
        out_specs=[
            pl.BlockSpec((_TG,), lambda m: (m,)),
            pl.BlockSpec((_TG,), lambda m: (m,)),
            pl.BlockSpec((_TG,), lambda m: (m,)),
            pl.BlockSpec((_TG,), lambda m: (m,)),
        ],
        out_shape=[
            jax.ShapeDtypeStruct((_B,), jnp.int32),
            jax.ShapeDtypeStruct((_B,), jnp.int32),
            jax.ShapeDtypeStruct((_B,), jnp.float32),
            jax.ShapeDtypeStruct((_B,), jnp.float32),
        ],
    )(x, wg_pad)


def _ffn_body(e_map_ref, x_ref, w1_ref, b1_ref, w2_ref, b2_ref, g_ref, y_ref):
    h = jnp.dot(x_ref[...], w1_ref[0], preferred_element_type=jnp.float32)
    h = jnp.maximum(h + b1_ref[0], 0.0).astype(jnp.bfloat16)
    y = jnp.dot(h, w2_ref[0], preferred_element_type=jnp.float32)
    y_ref[...] = ((y + b2_ref[0]) * g_ref[0, 0][:, None]).astype(jnp.bfloat16)


def _ffn(e_map, xg, w1, b1, w2, b2, g3):
    grid_spec = pltpu.PrefetchScalarGridSpec(
        num_scalar_prefetch=1,
        grid=(_NTILES,),
        in_specs=[
            pl.BlockSpec((_TM, _D), lambda m, em: (m, 0)),
            pl.BlockSpec((1, _D, _H), lambda m, em: (em[m], 0, 0)),
            pl.BlockSpec((1, 1, _H), lambda m, em: (em[m], 0, 0)),
            pl.BlockSpec((1, _H, _O), lambda m, em: (em[m], 0, 0)),
            pl.BlockSpec((1, 1, _O), lambda m, em: (em[m], 0, 0)),
            pl.BlockSpec((1, 1, _TM), lambda m, em: (m, 0, 0)),
        ],
        out_specs=pl.BlockSpec((_TM, _O), lambda m, em: (m, 0)),
    )
    return pl.pallas_call(
        _ffn_body,
        grid_spec=grid_spec,
        out_shape=jax.ShapeDtypeStruct((_PMAX, _O), jnp.bfloat16),
        compiler_params=pltpu.CompilerParams(
            dimension_semantics=("arbitrary",),
        ),
    )(e_map, xg, w1, b1, w2, b2, g3)


def kernel(x, Wg, W1, b1, W2, b2):
    wg_pad = jnp.zeros((_D, 128), jnp.float32).at[:, :_E].set(Wg)
    i1, i2, gw1, gw2 = _gate(x, wg_pad)

    eflat = jnp.stack([i1, i2], axis=1).reshape(-1)
    wflat = jnp.stack([gw1, gw2], axis=1).reshape(-1)
    onehot = (eflat[:, None] == jnp.arange(_E, dtype=jnp.int32)[None, :])
    cum = jnp.cumsum(onehot.astype(jnp.int32), axis=0)
    rank = jnp.take_along_axis(cum, eflat[:, None], axis=1)[:, 0] - 1
    counts = cum[-1]
    padded = ((counts + _TM - 1) // _TM) * _TM
    offsets = jnp.concatenate([jnp.zeros(1, padded.dtype), jnp.cumsum(padded)])
    pos = (offsets[eflat] + rank).astype(jnp.int32)
    p0, p1 = pos[0::2], pos[1::2]
    entry = jnp.arange(_B * _K, dtype=jnp.int32)
    tok = jnp.zeros(_PMAX, jnp.int32).at[pos].set(entry // _K)
    gwt = jnp.zeros(_PMAX, jnp.float32).at[pos].set(wflat)
    tile_start = jnp.arange(_NTILES, dtype=offsets.dtype) * _TM
    e_map = jnp.minimum(
        jnp.searchsorted(offsets[1:], tile_start, side="right"), _E - 1
    ).astype(jnp.int32)

    xg = jnp.take(x.astype(jnp.bfloat16), tok, axis=0)
    g3 = gwt.reshape(_NTILES, 1, _TM)
    y = _ffn(e_map, xg, W1.astype(jnp.bfloat16), b1[:, None, :],
             W2.astype(jnp.bfloat16), b2[:, None, :], g3)

    return (jnp.take(y, p0, axis=0).astype(jnp.float32)
            + jnp.take(y, p1, axis=0).astype(jnp.float32))


# speedup vs baseline: 1.3469x; 1.3469x over previous
import functools

import jax
import jax.numpy as jnp
from jax.experimental import pallas as pl
from jax.experimental.pallas import tpu as pltpu

_B, _D, _O, _H, _E, _K = 8192, 1024, 1024, 2048, 8, 2
_TM = 128
_PMAX = _B * _K + _E * _TM
_NTILES = _PMAX // _TM
_TG = 1024


def _gate_body(x_ref, wg_ref, i1_ref, i2_ref, w1_ref, w2_ref):
    s = jnp.dot(x_ref[...], wg_ref[...], preferred_element_type=jnp.float32)
    cols = jax.lax.broadcasted_iota(jnp.int32, s.shape, 1)
    neg = jnp.float32(-jnp.inf)
    s = jnp.where(cols < _E, s, neg)
    m1 = jnp.max(s, axis=1)
    i1 = jnp.min(jnp.where(s == m1[:, None], cols, _E), axis=1)
    s2 = jnp.where(cols == i1[:, None], neg, s)
    m2 = jnp.max(s2, axis=1)
    i2 = jnp.min(jnp.where(s2 == m2[:, None], cols, _E), axis=1)
    d = jnp.exp(m2 - m1)
    i1_ref[...] = i1
    i2_ref[...] = i2
    w1_ref[...] = 1.0 / (1.0 + d)
    w2_ref[...] = d / (1.0 + d)


def _gate(x, wg_pad):
    return pl.pallas_call(
        _gate_body,
        grid=(_B // _TG,),
        in_specs=[
            pl.BlockSpec((_TG, _D), lambda m: (m, 0)),
            pl.BlockSpec((_D, 128), lambda m: (0, 0)),
        ],
        out_specs=[
            pl.BlockSpec((_TG,), lambda m: (m,)),
            pl.BlockSpec((_TG,), lambda m: (m,)),
            pl.BlockSpec((_TG,), lambda m: (m,)),
            pl.BlockSpec((_TG,), lambda m: (m,)),
        ],
        out_shape=[
            jax.ShapeDtypeStruct((_B,), jnp.int32),
            jax.ShapeDtypeStruct((_B,), jnp.int32),
            jax.ShapeDtypeStruct((_B,), jnp.float32),
            jax.ShapeDtypeStruct((_B,), jnp.float32),
        ],
    )(x, wg_pad)


def _ffn_body(e_map_ref, x_ref, w1_ref, b1_ref, w2_ref, b2_ref, g_ref, y_ref):
    h = jnp.dot(x_ref[...].astype(jnp.bfloat16), w1_ref[0],
                preferred_element_type=jnp.float32)
    h = jnp.maximum(h + b1_ref[0], 0.0).astype(jnp.bfloat16)
    y = jnp.dot(h, w2_ref[0], preferred_element_type=jnp.float32)
    y_ref[...] = (y + b2_ref[0]) * g_ref[0, 0][:, None]


def _ffn(e_map, xg, w1, b1, w2, b2, g3):
    grid_spec = pltpu.PrefetchScalarGridSpec(
        num_scalar_prefetch=1,
        grid=(_NTILES,),
        in_specs=[
            pl.BlockSpec((_TM, _D), lambda m, em: (m, 0)),
            pl.BlockSpec((1, _D, _H), lambda m, em: (em[m], 0, 0)),
            pl.BlockSpec((1, 1, _H), lambda m, em: (em[m], 0, 0)),
            pl.BlockSpec((1, _H, _O), lambda m, em: (em[m], 0, 0)),
            pl.BlockSpec((1, 1, _O), lambda m, em: (em[m], 0, 0)),
            pl.BlockSpec((1, 1, _TM), lambda m, em: (m, 0, 0)),
        ],
        out_specs=pl.BlockSpec((_TM, _O), lambda m, em: (m, 0)),
    )
    return pl.pallas_call(
        _ffn_body,
        grid_spec=grid_spec,
        out_shape=jax.ShapeDtypeStruct((_PMAX, _O), jnp.float32),
        compiler_params=pltpu.CompilerParams(
            dimension_semantics=("arbitrary",),
        ),
    )(e_map, xg, w1, b1, w2, b2, g3)


def kernel(x, Wg, W1, b1, W2, b2):
    wg_pad = jnp.zeros((_D, 128), jnp.float32).at[:, :_E].set(Wg)
    i1, i2, gw1, gw2 = _gate(x, wg_pad)

    eflat = jnp.stack([i1, i2], axis=1).reshape(-1)
    wflat = jnp.stack([gw1, gw2], axis=1).reshape(-1)
    onehot = (eflat[:, None] == jnp.arange(_E, dtype=jnp.int32)[None, :])
    cum = jnp.cumsum(onehot.astype(jnp.int32), axis=0)
    rank = jnp.take_along_axis(cum, eflat[:, None], axis=1)[:, 0] - 1
    counts = cum[-1]
    padded = ((counts + _TM - 1) // _TM) * _TM
    offsets = jnp.concatenate([jnp.zeros(1, padded.dtype), jnp.cumsum(padded)])
    pos = (offsets[eflat] + rank).astype(jnp.int32)
    p0, p1 = pos[0::2], pos[1::2]
    entry = jnp.arange(_B * _K, dtype=jnp.int32)
    tok = jnp.zeros(_PMAX, jnp.int32).at[pos].set(entry // _K)
    gwt = jnp.zeros(_PMAX, jnp.float32).at[pos].set(wflat)
    tile_start = jnp.arange(_NTILES, dtype=offsets.dtype) * _TM
    e_map = jnp.minimum(
        jnp.searchsorted(offsets[1:], tile_start, side="right"), _E - 1
    ).astype(jnp.int32)

    xg = jnp.take(x, tok, axis=0)
    g3 = gwt.reshape(_NTILES, 1, _TM)
    y = _ffn(e_map, xg, W1.astype(jnp.bfloat16), b1[:, None, :],
             W2.astype(jnp.bfloat16), b2[:, None, :], g3)

    return jnp.take(y, p0, axis=0) + jnp.take(y, p1, axis=0)


# speedup vs baseline: 1.5643x; 1.1615x over previous
import functools

import jax
import jax.numpy as jnp
from jax.experimental import pallas as pl
from jax.experimental.pallas import tpu as pltpu

_B, _D, _O, _H, _E, _K = 8192, 1024, 1024, 2048, 8, 2
_TM = 128
_PMAX = _B * _K + _E * _TM
_NTILES = _PMAX // _TM
_TG = 1024


def _gate_body(x_ref, wg_ref, i1_ref, i2_ref, w1_ref, w2_ref):
    s = jnp.dot(x_ref[...], wg_ref[...], preferred_element_type=jnp.float32)
    cols = jax.lax.broadcasted_iota(jnp.int32, s.shape, 1)
    neg = jnp.float32(-jnp.inf)
    s = jnp.where(cols < _E, s, neg)
    m1 = jnp.max(s, axis=1)
    i1 = jnp.min(jnp.where(s == m1[:, None], cols, _E), axis=1)
    s2 = jnp.where(cols == i1[:, None], neg, s)
    m2 = jnp.max(s2, axis=1)
    i2 = jnp.min(jnp.where(s2 == m2[:, None], cols, _E), axis=1)
    d = jnp.exp(m2 - m1)
    i1_ref[...] = i1
    i2_ref[...] = i2
    w1_ref[...] = 1.0 / (1.0 + d)
    w2_ref[...] = d / (1.0 + d)


def _gate(x, wg_pad):
    return pl.pallas_call(
        _gate_body,
        grid=(_B // _TG,),
        in_specs=[
            pl.BlockSpec((_TG, _D), lambda m: (m, 0)),
            pl.BlockSpec((_D, 128), lambda m: (0, 0)),
        ],
        out_specs=[
            pl.BlockSpec((_TG,), lambda m: (m,)),
            pl.BlockSpec((_TG,), lambda m: (m,)),
            pl.BlockSpec((_TG,), lambda m: (m,)),
            pl.BlockSpec((_TG,), lambda m: (m,)),
        ],
        out_shape=[
            jax.ShapeDtypeStruct((_B,), jnp.int32),
            jax.ShapeDtypeStruct((_B,), jnp.int32),
            jax.ShapeDtypeStruct((_B,), jnp.float32),
            jax.ShapeDtypeStruct((_B,), jnp.float32),
        ],
    )(x, wg_pad)


def _ffn_body(e_map_ref, x_ref, w1_ref, b1_ref, w2_ref, b2_ref, g_ref, y_ref):
    h = jnp.dot(x_ref[...].astype(jnp.bfloat16), w1_ref[0],
                preferred_element_type=jnp.float32)
    h = jnp.maximum(h + b1_ref[0], 0.0).astype(jnp.bfloat16)
    y = jnp.dot(h, w2_ref[0], preferred_element_type=jnp.float32)
    y_ref[...] = (y + b2_ref[0]) * g_ref[0, 0][:, None]


def _ffn(e_map, xg, w1, b1, w2, b2, g3):
    grid_spec = pltpu.PrefetchScalarGridSpec(
        num_scalar_prefetch=1,
        grid=(_NTILES,),
        in_specs=[
            pl.BlockSpec((_TM, _D), lambda m, em: (m, 0)),
            pl.BlockSpec((1, _D, _H), lambda m, em: (em[m], 0, 0)),
            pl.BlockSpec((1, 1, _H), lambda m, em: (em[m], 0, 0)),
            pl.BlockSpec((1, _H, _O), lambda m, em: (em[m], 0, 0)),
            pl.BlockSpec((1, 1, _O), lambda m, em: (em[m], 0, 0)),
            pl.BlockSpec((1, 1, _TM), lambda m, em: (m, 0, 0)),
        ],
        out_specs=pl.BlockSpec((_TM, _O), lambda m, em: (m, 0)),
    )
    return pl.pallas_call(
        _ffn_body,
        grid_spec=grid_spec,
        out_shape=jax.ShapeDtypeStruct((_PMAX, _O), jnp.float32),
        compiler_params=pltpu.CompilerParams(
            dimension_semantics=("arbitrary",),
        ),
    )(e_map, xg, w1, b1, w2, b2, g3)


def kernel(x, Wg, W1, b1, W2, b2):
    wg_pad = jnp.zeros((_D, 128), jnp.float32).at[:, :_E].set(Wg)
    i1, i2, gw1, gw2 = _gate(x, wg_pad)

    eflat = jnp.stack([i1, i2], axis=1).reshape(-1)
    wflat = jnp.stack([gw1, gw2], axis=1).reshape(-1)
    onehot = (eflat[:, None] == jnp.arange(_E, dtype=jnp.int32)[None, :])
    cum = jnp.cumsum(onehot.astype(jnp.int32), axis=0)
    rank = jnp.take_along_axis(cum, eflat[:, None], axis=1)[:, 0] - 1
    counts = cum[-1]
    padded = ((counts + _TM - 1) // _TM) * _TM
    offsets = jnp.concatenate([jnp.zeros(1, padded.dtype), jnp.cumsum(padded)])
    pos = (offsets[eflat] + rank).astype(jnp.int32)
    p0, p1 = pos[0::2], pos[1::2]
    entry = jnp.arange(_B * _K, dtype=jnp.int32)
    packed = jnp.stack([entry // _K, jax.lax.bitcast_convert_type(
        wflat, jnp.int32)], axis=1)
    grouped = jnp.zeros((_PMAX, 2), jnp.int32).at[pos].set(
        packed, mode="promise_in_bounds", unique_indices=True)
    tok = grouped[:, 0]
    gwt = jax.lax.bitcast_convert_type(grouped[:, 1], jnp.float32)
    tile_start = jnp.arange(_NTILES, dtype=offsets.dtype) * _TM
    e_map = jnp.minimum(
        jnp.searchsorted(offsets[1:], tile_start, side="right"), _E - 1
    ).astype(jnp.int32)

    xg = x.at[tok].get(mode="promise_in_bounds")
    g3 = gwt.reshape(_NTILES, 1, _TM)
    y = _ffn(e_map, xg, W1.astype(jnp.bfloat16), b1[:, None, :],
             W2.astype(jnp.bfloat16), b2[:, None, :], g3)

    return (y.at[p0].get(mode="promise_in_bounds")
            + y.at[p1].get(mode="promise_in_bounds"))


# speedup vs baseline: 1.6661x; 1.0650x over previous
import functools

import jax
import jax.numpy as jnp
from jax.experimental import pallas as pl
from jax.experimental.pallas import tpu as pltpu

_B, _D, _O, _H, _E, _K = 8192, 1024, 1024, 2048, 8, 2
_TM = 128
_PMAX = _B * _K + _E * _TM
_NTILES = _PMAX // _TM
_TG = 1024


def _gate_body(x_ref, wg_ref, i1_ref, i2_ref, w1_ref, w2_ref):
    s = jnp.dot(x_ref[...], wg_ref[...], preferred_element_type=jnp.float32)
    cols = jax.lax.broadcasted_iota(jnp.int32, s.shape, 1)
    neg = jnp.float32(-jnp.inf)
    s = jnp.where(cols < _E, s, neg)
    m1 = jnp.max(s, axis=1)
    i1 = jnp.min(jnp.where(s == m1[:, None], cols, _E), axis=1)
    s2 = jnp.where(cols == i1[:, None], neg, s)
    m2 = jnp.max(s2, axis=1)
    i2 = jnp.min(jnp.where(s2 == m2[:, None], cols, _E), axis=1)
    d = jnp.exp(m2 - m1)
    i1_ref[...] = i1
    i2_ref[...] = i2
    w1_ref[...] = 1.0 / (1.0 + d)
    w2_ref[...] = d / (1.0 + d)


def _gate(x, wg_pad):
    return pl.pallas_call(
        _gate_body,
        grid=(_B // _TG,),
        in_specs=[
            pl.BlockSpec((_TG, _D), lambda m: (m, 0)),
            pl.BlockSpec((_D, 128), lambda m: (0, 0)),
        ],
        out_specs=[
            pl.BlockSpec((_TG,), lambda m: (m,)),
            pl.BlockSpec((_TG,), lambda m: (m,)),
            pl.BlockSpec((_TG,), lambda m: (m,)),
            pl.BlockSpec((_TG,), lambda m: (m,)),
        ],
        out_shape=[
            jax.ShapeDtypeStruct((_B,), jnp.int32),
            jax.ShapeDtypeStruct((_B,), jnp.int32),
            jax.ShapeDtypeStruct((_B,), jnp.float32),
            jax.ShapeDtypeStruct((_B,), jnp.float32),
        ],
    )(x, wg_pad)


def _ffn_body(e_map_ref, x_ref, w1_ref, b1_ref, w2_ref, b2_ref, g_ref, y_ref):
    h = jnp.dot(x_ref[...], w1_ref[0], precision=jax.lax.Precision.DEFAULT,
                preferred_element_type=jnp.float32)
    h = jnp.maximum(h + b1_ref[0], 0.0)
    y = jnp.dot(h, w2_ref[0], precision=jax.lax.Precision.DEFAULT,
                preferred_element_type=jnp.float32)
    y_ref[...] = (y + b2_ref[0]) * g_ref[0, 0][:, None]


def _ffn(e_map, xg, w1, b1, w2, b2, g3):
    grid_spec = pltpu.PrefetchScalarGridSpec(
        num_scalar_prefetch=1,
        grid=(_NTILES,),
        in_specs=[
            pl.BlockSpec((_TM, _D), lambda m, em: (m, 0)),
            pl.BlockSpec((1, _D, _H), lambda m, em: (em[m], 0, 0)),
            pl.BlockSpec((1, 1, _H), lambda m, em: (em[m], 0, 0)),
            pl.BlockSpec((1, _H, _O), lambda m, em: (em[m], 0, 0)),
            pl.BlockSpec((1, 1, _O), lambda m, em: (em[m], 0, 0)),
            pl.BlockSpec((1, 1, _TM), lambda m, em: (m, 0, 0)),
        ],
        out_specs=pl.BlockSpec((_TM, _O), lambda m, em: (m, 0)),
    )
    return pl.pallas_call(
        _ffn_body,
        grid_spec=grid_spec,
        out_shape=jax.ShapeDtypeStruct((_PMAX, _O), jnp.float32),
        compiler_params=pltpu.CompilerParams(
            dimension_semantics=("arbitrary",),
        ),
    )(e_map, xg, w1, b1, w2, b2, g3)


def kernel(x, Wg, W1, b1, W2, b2):
    wg_pad = jnp.zeros((_D, 128), jnp.float32).at[:, :_E].set(Wg)
    i1, i2, gw1, gw2 = _gate(x, wg_pad)

    eflat = jnp.stack([i1, i2], axis=1).reshape(-1)
    wflat = jnp.stack([gw1, gw2], axis=1).reshape(-1)
    onehot = (eflat[:, None] == jnp.arange(_E, dtype=jnp.int32)[None, :])
    cum = jnp.cumsum(onehot.astype(jnp.int32), axis=0)
    rank = jnp.take_along_axis(cum, eflat[:, None], axis=1)[:, 0] - 1
    counts = cum[-1]
    padded = ((counts + _TM - 1) // _TM) * _TM
    offsets = jnp.concatenate([jnp.zeros(1, padded.dtype), jnp.cumsum(padded)])
    pos = (offsets[eflat] + rank).astype(jnp.int32)
    p0, p1 = pos[0::2], pos[1::2]
    entry = jnp.arange(_B * _K, dtype=jnp.int32)
    packed = jnp.stack([entry // _K, jax.lax.bitcast_convert_type(
        wflat, jnp.int32)], axis=1)
    grouped = jnp.zeros((_PMAX, 2), jnp.int32).at[pos].set(
        packed, mode="promise_in_bounds", unique_indices=True)
    tok = grouped[:, 0]
    gwt = jax.lax.bitcast_convert_type(grouped[:, 1], jnp.float32)
    tile_start = jnp.arange(_NTILES, dtype=offsets.dtype) * _TM
    e_map = jnp.minimum(
        jnp.searchsorted(offsets[1:], tile_start, side="right"), _E - 1
    ).astype(jnp.int32)

    xg = x.at[tok].get(mode="promise_in_bounds")
    g3 = gwt.reshape(_NTILES, 1, _TM)
    y = _ffn(e_map, xg, W1, b1[:, None, :],
             W2, b2[:, None, :], g3)

    return (y.at[p0].get(mode="promise_in_bounds")
            + y.at[p1].get(mode="promise_in_bounds"))


# speedup vs baseline: 1.7212x; 1.0331x over previous
import functools

import jax
import jax.numpy as jnp
from jax.experimental import pallas as pl
from jax.experimental.pallas import tpu as pltpu

_B, _D, _O, _H, _E, _K = 8192, 1024, 1024, 2048, 8, 2
_TM = 256
_PMAX = _B * _K + _E * _TM
_NTILES = _PMAX // _TM
_TG = 1024


def _gate_body(x_ref, wg_ref, i1_ref, i2_ref, w1_ref, w2_ref):
    s = jnp.dot(x_ref[...], wg_ref[...], preferred_element_type=jnp.float32)
    cols = jax.lax.broadcasted_iota(jnp.int32, s.shape, 1)
    neg = jnp.float32(-jnp.inf)
    s = jnp.where(cols < _E, s, neg)
    m1 = jnp.max(s, axis=1)
    i1 = jnp.min(jnp.where(s == m1[:, None], cols, _E), axis=1)
    s2 = jnp.where(cols == i1[:, None], neg, s)
    m2 = jnp.max(s2, axis=1)
    i2 = jnp.min(jnp.where(s2 == m2[:, None], cols, _E), axis=1)
    d = jnp.exp(m2 - m1)
    i1_ref[...] = i1
    i2_ref[...] = i2
    w1_ref[...] = 1.0 / (1.0 + d)
    w2_ref[...] = d / (1.0 + d)


def _gate(x, wg_pad):
    return pl.pallas_call(
        _gate_body,
        grid=(_B // _TG,),
        in_specs=[
            pl.BlockSpec((_TG, _D), lambda m: (m, 0)),
            pl.BlockSpec((_D, 128), lambda m: (0, 0)),
        ],
        out_specs=[
            pl.BlockSpec((_TG,), lambda m: (m,)),
            pl.BlockSpec((_TG,), lambda m: (m,)),
            pl.BlockSpec((_TG,), lambda m: (m,)),
            pl.BlockSpec((_TG,), lambda m: (m,)),
        ],
        out_shape=[
            jax.ShapeDtypeStruct((_B,), jnp.int32),
            jax.ShapeDtypeStruct((_B,), jnp.int32),
            jax.ShapeDtypeStruct((_B,), jnp.float32),
            jax.ShapeDtypeStruct((_B,), jnp.float32),
        ],
    )(x, wg_pad)


def _ffn_body(e_map_ref, x_ref, w1_ref, b1_ref, w2_ref, b2_ref, g_ref, y_ref):
    h = jnp.dot(x_ref[...], w1_ref[0], precision=jax.lax.Precision.DEFAULT,
                preferred_element_type=jnp.float32)
    h = jnp.maximum(h + b1_ref[0], 0.0)
    y = jnp.dot(h, w2_ref[0], precision=jax.lax.Precision.DEFAULT,
                preferred_element_type=jnp.float32)
    y_ref[...] = (y + b2_ref[0]) * g_ref[0, 0][:, None]


def _ffn(e_map, xg, w1, b1, w2, b2, g3):
    grid_spec = pltpu.PrefetchScalarGridSpec(
        num_scalar_prefetch=1,
        grid=(_NTILES,),
        in_specs=[
            pl.BlockSpec((_TM, _D), lambda m, em: (m, 0)),
            pl.BlockSpec((1, _D, _H), lambda m, em: (em[m], 0, 0)),
            pl.BlockSpec((1, 1, _H), lambda m, em: (em[m], 0, 0)),
            pl.BlockSpec((1, _H, _O), lambda m, em: (em[m], 0, 0)),
            pl.BlockSpec((1, 1, _O), lambda m, em: (em[m], 0, 0)),
            pl.BlockSpec((1, 1, _TM), lambda m, em: (m, 0, 0)),
        ],
        out_specs=pl.BlockSpec((_TM, _O), lambda m, em: (m, 0)),
    )
    return pl.pallas_call(
        _ffn_body,
        grid_spec=grid_spec,
        out_shape=jax.ShapeDtypeStruct((_PMAX, _O), jnp.float32),
        compiler_params=pltpu.CompilerParams(
            dimension_semantics=("arbitrary",),
        ),
    )(e_map, xg, w1, b1, w2, b2, g3)


def kernel(x, Wg, W1, b1, W2, b2):
    wg_pad = jnp.zeros((_D, 128), jnp.float32).at[:, :_E].set(Wg)
    i1, i2, gw1, gw2 = _gate(x, wg_pad)

    eflat = jnp.stack([i1, i2], axis=1).reshape(-1)
    wflat = jnp.stack([gw1, gw2], axis=1).reshape(-1)
    onehot = (eflat[:, None] == jnp.arange(_E, dtype=jnp.int32)[None, :])
    cum = jnp.cumsum(onehot.astype(jnp.int32), axis=0)
    rank = jnp.take_along_axis(cum, eflat[:, None], axis=1)[:, 0] - 1
    counts = cum[-1]
    padded = ((counts + _TM - 1) // _TM) * _TM
    offsets = jnp.concatenate([jnp.zeros(1, padded.dtype), jnp.cumsum(padded)])
    pos = (offsets[eflat] + rank).astype(jnp.int32)
    p0, p1 = pos[0::2], pos[1::2]
    entry = jnp.arange(_B * _K, dtype=jnp.int32)
    packed = jnp.stack([entry // _K, jax.lax.bitcast_convert_type(
        wflat, jnp.int32)], axis=1)
    grouped = jnp.zeros((_PMAX, 2), jnp.int32).at[pos].set(
        packed, mode="promise_in_bounds", unique_indices=True)
    tok = grouped[:, 0]
    gwt = jax.lax.bitcast_convert_type(grouped[:, 1], jnp.float32)
    tile_start = jnp.arange(_NTILES, dtype=offsets.dtype) * _TM
    e_map = jnp.minimum(
        jnp.searchsorted(offsets[1:], tile_start, side="right"), _E - 1
    ).astype(jnp.int32)

    xg = x.at[tok].get(mode="promise_in_bounds")
    g3 = gwt.reshape(_NTILES, 1, _TM)
    y = _ffn(e_map, xg, W1, b1[:, None, :],
             W2, b2[:, None, :], g3)

    return (y.at[p0].get(mode="promise_in_bounds")
            + y.at[p1].get(mode="promise_in_bounds"))
